# baseline (speedup 1.0000x reference)
import jax
import jax.numpy as jnp
from jax import lax
from jax.experimental import pallas as pl
from jax.experimental.pallas import tpu as pltpu
from jax.experimental.pallas import tpu_sc as plsc

B, D, C = 512, 32, 64
ROWS = 128
STEPS = B // ROWS
NEG_INF = float("-inf")

_NT = (((1,), (1,)), ((), ()))
_NN = (((1,), (0,)), ((), ()))

NSUB = 16
RPS = B // NSUB
L = 16


def _sc_mine_body(lab_hbm, logit_hbm, laboh_hbm, ccoh_hbm, stats_hbm,
                  logit_v, lab_v, laboh_v, ccoh_v, w_v, cnt_v,
                  part_v, all_v, shared):
    sid = lax.axis_index("s")
    base = sid * RPS

    pltpu.sync_copy(lab_hbm.at[pl.ds(base, RPS)], lab_v)
    pltpu.sync_copy(logit_hbm.at[pl.ds(base, RPS)], logit_v)

    lane = lax.iota(jnp.int32, L)

    def allmax(v):
        for sh in (8, 4, 2, 1):
            v = jnp.maximum(v, v[lane ^ sh])
        return v

    def allmin(v):
        for sh in (8, 4, 2, 1):
            v = jnp.minimum(v, v[lane ^ sh])
        return v

    cc_acc = [jnp.zeros((L,), jnp.int32), jnp.zeros((L,), jnp.int32)]
    h = [jnp.zeros((L,), jnp.float32) for _ in range(C // L)]
    n = [jnp.zeros((L,), jnp.float32) for _ in range(C // L)]
    labq = [lab_v[pl.ds(0, L)], lab_v[pl.ds(L, L)]]
    for r in range(RPS):
        m = logit_v[r, pl.ds(0, L)]
        ch = jnp.zeros((L,), jnp.int32)
        for k in range(1, C // L):
            ck = logit_v[r, pl.ds(k * L, L)]
            g = ck > m
            m = jnp.where(g, ck, m)
            ch = jnp.where(g, k, ch)
        gmax = allmax(m)
        cand = ch * L + lane
        ccr16 = allmin(jnp.where(m == gmax, cand, C))
        q, r16 = divmod(r, L)
        cc_acc[q] = jnp.where(lane == r16, ccr16, cc_acc[q])
        ccr = ccr16[0]
        labr = labq[q][r16]
        for k in range(C // L):
            cls = lane + k * L
            ohl = jnp.where(cls == labr, 1.0, 0.0)
            ohc = jnp.where(cls == ccr, 1.0, 0.0)
            h[k] = h[k] + ohl
            n[k] = n[k] + ohc
            laboh_v[r, pl.ds(k * L, L)] = ohl
            ccoh_v[r, pl.ds(k * L, L)] = ohc

    for k in range(C // L):
        part_v[pl.ds(k * L, L)] = h[k]
        part_v[pl.ds(C + k * L, L)] = n[k]
    pltpu.sync_copy(part_v, shared.at[sid])
    plsc.subcore_barrier()
    pltpu.sync_copy(shared, all_v)
    hist = []
    cnt = []
    for k in range(C // L):
        acc = all_v[0, pl.ds(k * L, L)]
        accn = all_v[0, pl.ds(C + k * L, L)]
        for v in range(1, NSUB):
            acc = acc + all_v[v, pl.ds(k * L, L)]
            accn = accn + all_v[v, pl.ds(C + k * L, L)]
        hist.append(acc)
        cnt.append(accn)

    for q in range(RPS // L):
        idx = cc_acc[q]
        off = idx & (L - 1)
        chunk = idx >> 4
        g = hist[0][off]
        for k in range(1, C // L):
            g = jnp.where(chunk == k, hist[k][off], g)
        w_v[pl.ds(q * L, L)] = g

    pltpu.sync_copy(laboh_v, laboh_hbm.at[pl.ds(base, RPS)])
    pltpu.sync_copy(ccoh_v, ccoh_hbm.at[pl.ds(base, RPS)])
    pltpu.sync_copy(w_v, stats_hbm.at[0, pl.ds(base, RPS)])

    @pl.when(sid == 0)
    def _cnt_row():
        for k in range(C // L):
            cnt_v[pl.ds(k * L, L)] = cnt[k]
        pltpu.sync_copy(cnt_v, stats_hbm.at[1, pl.ds(0, C)])


def _sc_mine(labels, logits):
    mesh = plsc.VectorSubcoreMesh(core_axis_name="c", subcore_axis_name="s",
                                  num_cores=1)
    return pl.kernel(
        _sc_mine_body,
        out_type=[jax.ShapeDtypeStruct((B, C), jnp.float32),
                  jax.ShapeDtypeStruct((B, C), jnp.float32),
                  jax.ShapeDtypeStruct((8, B), jnp.float32)],
        mesh=mesh,
        scratch_types=[
            pltpu.VMEM((RPS, C), jnp.float32),
            pltpu.VMEM((RPS,), jnp.int32),
            pltpu.VMEM((RPS, C), jnp.float32),
            pltpu.VMEM((RPS, C), jnp.float32),
            pltpu.VMEM((RPS,), jnp.float32),
            pltpu.VMEM((C,), jnp.float32),
            pltpu.VMEM((2 * C,), jnp.float32),
            pltpu.VMEM((NSUB, 2 * C), jnp.float32),
            pltpu.VMEM_SHARED((NSUB, 2 * C), jnp.float32),
        ],
    )(labels, logits)


def _nce_body(src_ref, tgt_ref, laboh_ref, ccoh_ref, stats_ref, out_ref,
              lnw_ref, acc_ref):
    i = pl.program_id(0)

    @pl.when(i == 0)
    def _prep():
        w = stats_ref[pl.ds(0, 1), :]
        lnw_ref[:, :] = jnp.where(w > 0.0, jnp.log(w), NEG_INF)
        acc_ref[0] = 0.0
        acc_ref[1] = 0.0
        acc_ref[2] = jnp.sum(w)

    src = src_ref[pl.ds(i * ROWS, ROWS), :]
    lab_oh = laboh_ref[pl.ds(i * ROWS, ROWS), :]
    s = lax.dot_general(src, tgt_ref[:], _NT,
                        preferred_element_type=jnp.float32)

    t = s + lnw_ref[:, :]
    smax = jnp.max(t, axis=1, keepdims=True)
    e = jnp.exp(t - smax)
    sumexp = jnp.sum(e, axis=1, keepdims=True)
    lse = smax + jnp.log(sumexp)

    G = lax.dot_general(s, ccoh_ref[:, :], _NN,
                        preferred_element_type=jnp.float32)
    cnt = stats_ref[pl.ds(1, 1), pl.ds(0, C)]
    rowcnt = lax.dot_general(lab_oh, cnt, _NT,
                             preferred_element_type=jnp.float32)

    acc_ref[0] += jnp.sum(lab_oh * G)
    acc_ref[1] += jnp.sum(jnp.where(rowcnt > 0.0, rowcnt * lse, 0.0))

    @pl.when(i == STEPS - 1)
    def _finish():
        nce = (acc_ref[0] - acc_ref[1]) / (-1.0 * B * acc_ref[2])
        out_ref[:, :] = jnp.reshape(nce, (1, 1))


@jax.jit
def kernel(src_feas, src_labels, tgt_feas, tgt_logits):
    labels = src_labels.astype(jnp.int32)
    lab_oh, cc_oh, stats = _sc_mine(labels, tgt_logits)

    out = pl.pallas_call(
        _nce_body,
        grid=(STEPS,),
        in_specs=[
            pl.BlockSpec((B, D), lambda i: (0, 0)),
            pl.BlockSpec((B, D), lambda i: (0, 0)),
            pl.BlockSpec((B, C), lambda i: (0, 0)),
            pl.BlockSpec((B, C), lambda i: (0, 0)),
            pl.BlockSpec((8, B), lambda i: (0, 0)),
        ],
        out_specs=pl.BlockSpec((1, 1), lambda i: (0, 0)),
        out_shape=jax.ShapeDtypeStruct((1, 1), jnp.float32),
        scratch_shapes=[
            pltpu.VMEM((1, B), jnp.float32),
            pltpu.SMEM((4,), jnp.float32),
        ],
    )(src_feas, tgt_feas, lab_oh, cc_oh, stats)
    return out[0, 0]


# speedup vs baseline: 1.0379x; 1.0379x over previous
import jax
import jax.numpy as jnp
from jax import lax
from jax.experimental import pallas as pl
from jax.experimental.pallas import tpu as pltpu
from jax.experimental.pallas import tpu_sc as plsc

B, D, C = 512, 32, 64
ROWS = 128
STEPS = B // ROWS
NEG_INF = float("-inf")

_NT = (((1,), (1,)), ((), ()))
_NN = (((1,), (0,)), ((), ()))

NSUB = 16
RPS = B // NSUB
L = 16


def _sc_mine_body(lab_hbm, logit_hbm, laboh_hbm, ccoh_hbm, stats_hbm,
                  logit_v, lab_v, laboh_v, ccoh_v, w_v, cnt_v,
                  part_v, all_v, shared, sem):
    sid = lax.axis_index("s")
    base = sid * RPS

    cin0 = pltpu.make_async_copy(lab_hbm.at[pl.ds(base, RPS)], lab_v, sem)
    cin1 = pltpu.make_async_copy(logit_hbm.at[pl.ds(base, RPS)], logit_v, sem)
    cin0.start()
    cin1.start()
    cin0.wait()
    cin1.wait()

    lane = lax.iota(jnp.int32, L)

    def allmax(v):
        for sh in (8, 4, 2, 1):
            v = jnp.maximum(v, v[lane ^ sh])
        return v

    def allmin(v):
        for sh in (8, 4, 2, 1):
            v = jnp.minimum(v, v[lane ^ sh])
        return v

    cc_acc = [jnp.zeros((L,), jnp.int32), jnp.zeros((L,), jnp.int32)]
    h = [jnp.zeros((L,), jnp.float32) for _ in range(C // L)]
    n = [jnp.zeros((L,), jnp.float32) for _ in range(C // L)]
    labq = [lab_v[pl.ds(0, L)], lab_v[pl.ds(L, L)]]
    for r in range(RPS):
        m = logit_v[r, pl.ds(0, L)]
        ch = jnp.zeros((L,), jnp.int32)
        for k in range(1, C // L):
            ck = logit_v[r, pl.ds(k * L, L)]
            g = ck > m
            m = jnp.where(g, ck, m)
            ch = jnp.where(g, k, ch)
        gmax = allmax(m)
        cand = ch * L + lane
        ccr16 = allmin(jnp.where(m == gmax, cand, C))
        q, r16 = divmod(r, L)
        cc_acc[q] = jnp.where(lane == r16, ccr16, cc_acc[q])
        ccr = ccr16[0]
        labr = labq[q][r16]
        for k in range(C // L):
            cls = lane + k * L
            ohl = jnp.where(cls == labr, 1.0, 0.0)
            ohc = jnp.where(cls == ccr, 1.0, 0.0)
            h[k] = h[k] + ohl
            n[k] = n[k] + ohc
            laboh_v[r, pl.ds(k * L, L)] = ohl
            ccoh_v[r, pl.ds(k * L, L)] = ohc

    cout0 = pltpu.make_async_copy(laboh_v, laboh_hbm.at[pl.ds(base, RPS)], sem)
    cout1 = pltpu.make_async_copy(ccoh_v, ccoh_hbm.at[pl.ds(base, RPS)], sem)
    cout0.start()
    cout1.start()

    for k in range(C // L):
        part_v[pl.ds(k * L, L)] = h[k]
        part_v[pl.ds(C + k * L, L)] = n[k]
    pltpu.sync_copy(part_v, shared.at[sid])
    plsc.subcore_barrier()
    pltpu.sync_copy(shared, all_v)
    hist = []
    cnt = []
    for k in range(C // L):
        acc = all_v[0, pl.ds(k * L, L)]
        accn = all_v[0, pl.ds(C + k * L, L)]
        for v in range(1, NSUB):
            acc = acc + all_v[v, pl.ds(k * L, L)]
            accn = accn + all_v[v, pl.ds(C + k * L, L)]
        hist.append(acc)
        cnt.append(accn)

    for q in range(RPS // L):
        idx = cc_acc[q]
        off = idx & (L - 1)
        chunk = idx >> 4
        g = hist[0][off]
        for k in range(1, C // L):
            g = jnp.where(chunk == k, hist[k][off], g)
        w_v[pl.ds(q * L, L)] = g

    pltpu.sync_copy(w_v, stats_hbm.at[0, pl.ds(base, RPS)])

    @pl.when(sid == 0)
    def _cnt_row():
        for k in range(C // L):
            cnt_v[pl.ds(k * L, L)] = cnt[k]
        pltpu.sync_copy(cnt_v, stats_hbm.at[1, pl.ds(0, C)])

    cout0.wait()
    cout1.wait()


def _sc_mine(labels, logits):
    mesh = plsc.VectorSubcoreMesh(core_axis_name="c", subcore_axis_name="s",
                                  num_cores=1)
    return pl.kernel(
        _sc_mine_body,
        out_type=[jax.ShapeDtypeStruct((B, C), jnp.float32),
                  jax.ShapeDtypeStruct((B, C), jnp.float32),
                  jax.ShapeDtypeStruct((8, B), jnp.float32)],
        mesh=mesh,
        scratch_types=[
            pltpu.VMEM((RPS, C), jnp.float32),
            pltpu.VMEM((RPS,), jnp.int32),
            pltpu.VMEM((RPS, C), jnp.float32),
            pltpu.VMEM((RPS, C), jnp.float32),
            pltpu.VMEM((RPS,), jnp.float32),
            pltpu.VMEM((C,), jnp.float32),
            pltpu.VMEM((2 * C,), jnp.float32),
            pltpu.VMEM((NSUB, 2 * C), jnp.float32),
            pltpu.VMEM_SHARED((NSUB, 2 * C), jnp.float32),
            pltpu.SemaphoreType.DMA,
        ],
    )(labels, logits)


def _nce_body(src_ref, tgt_ref, laboh_ref, ccoh_ref, stats_ref, out_ref,
              lnw_ref, acc_ref):
    i = pl.program_id(0)

    @pl.when(i == 0)
    def _prep():
        w = stats_ref[pl.ds(0, 1), :]
        lnw_ref[:, :] = jnp.where(w > 0.0, jnp.log(w), NEG_INF)
        acc_ref[0] = 0.0
        acc_ref[1] = 0.0
        acc_ref[2] = jnp.sum(w)

    src = src_ref[pl.ds(i * ROWS, ROWS), :]
    lab_oh = laboh_ref[pl.ds(i * ROWS, ROWS), :]
    s = lax.dot_general(src, tgt_ref[:], _NT,
                        preferred_element_type=jnp.float32)

    t = s + lnw_ref[:, :]
    smax = jnp.max(t, axis=1, keepdims=True)
    e = jnp.exp(t - smax)
    sumexp = jnp.sum(e, axis=1, keepdims=True)
    lse = smax + jnp.log(sumexp)

    G = lax.dot_general(s, ccoh_ref[:, :], _NN,
                        preferred_element_type=jnp.float32)
    cnt = stats_ref[pl.ds(1, 1), pl.ds(0, C)]
    rowcnt = lax.dot_general(lab_oh, cnt, _NT,
                             preferred_element_type=jnp.float32)

    acc_ref[0] += jnp.sum(lab_oh * G)
    acc_ref[1] += jnp.sum(jnp.where(rowcnt > 0.0, rowcnt * lse, 0.0))

    @pl.when(i == STEPS - 1)
    def _finish():
        nce = (acc_ref[0] - acc_ref[1]) / (-1.0 * B * acc_ref[2])
        out_ref[:, :] = jnp.reshape(nce, (1, 1))


@jax.jit
def kernel(src_feas, src_labels, tgt_feas, tgt_logits):
    labels = src_labels.astype(jnp.int32)
    lab_oh, cc_oh, stats = _sc_mine(labels, tgt_logits)

    out = pl.pallas_call(
        _nce_body,
        grid=(STEPS,),
        in_specs=[
            pl.BlockSpec((B, D), lambda i: (0, 0)),
            pl.BlockSpec((B, D), lambda i: (0, 0)),
            pl.BlockSpec((B, C), lambda i: (0, 0)),
            pl.BlockSpec((B, C), lambda i: (0, 0)),
            pl.BlockSpec((8, B), lambda i: (0, 0)),
        ],
        out_specs=pl.BlockSpec((1, 1), lambda i: (0, 0)),
        out_shape=jax.ShapeDtypeStruct((1, 1), jnp.float32),
        scratch_shapes=[
            pltpu.VMEM((1, B), jnp.float32),
            pltpu.SMEM((4,), jnp.float32),
        ],
    )(src_feas, tgt_feas, lab_oh, cc_oh, stats)
    return out[0, 0]


# speedup vs baseline: 1.0798x; 1.0404x over previous
import jax
import jax.numpy as jnp
from jax import lax
from jax.experimental import pallas as pl
from jax.experimental.pallas import tpu as pltpu
from jax.experimental.pallas import tpu_sc as plsc

B, D, C = 512, 32, 64
ROWS = 256
STEPS = B // ROWS
NEG_INF = float("-inf")

_NT = (((1,), (1,)), ((), ()))
_NN = (((1,), (0,)), ((), ()))

NSUB = 16
RPS = B // NSUB
L = 16


def _sc_mine_body(lab_hbm, logit_hbm, laboh_hbm, ccoh_hbm, stats_hbm,
                  logit_v, lab_v, laboh_v, ccoh_v, w_v, cnt_v,
                  part_v, all_v, shared, sem):
    sid = lax.axis_index("s")
    base = sid * RPS

    cin0 = pltpu.make_async_copy(lab_hbm.at[pl.ds(base, RPS)], lab_v, sem)
    cin1 = pltpu.make_async_copy(logit_hbm.at[pl.ds(base, RPS)], logit_v, sem)
    cin0.start()
    cin1.start()
    cin0.wait()
    cin1.wait()

    lane = lax.iota(jnp.int32, L)

    def allmax(v):
        for sh in (8, 4, 2, 1):
            v = jnp.maximum(v, v[lane ^ sh])
        return v

    def allmin(v):
        for sh in (8, 4, 2, 1):
            v = jnp.minimum(v, v[lane ^ sh])
        return v

    cc_acc = [jnp.zeros((L,), jnp.int32), jnp.zeros((L,), jnp.int32)]
    h = [jnp.zeros((L,), jnp.float32) for _ in range(C // L)]
    n = [jnp.zeros((L,), jnp.float32) for _ in range(C // L)]
    labq = [lab_v[pl.ds(0, L)], lab_v[pl.ds(L, L)]]
    for r in range(RPS):
        m = logit_v[r, pl.ds(0, L)]
        ch = jnp.zeros((L,), jnp.int32)
        for k in range(1, C // L):
            ck = logit_v[r, pl.ds(k * L, L)]
            g = ck > m
            m = jnp.where(g, ck, m)
            ch = jnp.where(g, k, ch)
        gmax = allmax(m)
        cand = ch * L + lane
        ccr16 = allmin(jnp.where(m == gmax, cand, C))
        q, r16 = divmod(r, L)
        cc_acc[q] = jnp.where(lane == r16, ccr16, cc_acc[q])
        ccr = ccr16[0]
        labr = labq[q][r16]
        for k in range(C // L):
            cls = lane + k * L
            ohl = jnp.where(cls == labr, 1.0, 0.0)
            ohc = jnp.where(cls == ccr, 1.0, 0.0)
            h[k] = h[k] + ohl
            n[k] = n[k] + ohc
            laboh_v[r, pl.ds(k * L, L)] = ohl
            ccoh_v[r, pl.ds(k * L, L)] = ohc

    cout0 = pltpu.make_async_copy(laboh_v, laboh_hbm.at[pl.ds(base, RPS)], sem)
    cout1 = pltpu.make_async_copy(ccoh_v, ccoh_hbm.at[pl.ds(base, RPS)], sem)
    cout0.start()
    cout1.start()

    for k in range(C // L):
        part_v[pl.ds(k * L, L)] = h[k]
        part_v[pl.ds(C + k * L, L)] = n[k]
    pltpu.sync_copy(part_v, shared.at[sid])
    plsc.subcore_barrier()
    pltpu.sync_copy(shared, all_v)
    hist = []
    cnt = []
    for k in range(C // L):
        acc = all_v[0, pl.ds(k * L, L)]
        accn = all_v[0, pl.ds(C + k * L, L)]
        for v in range(1, NSUB):
            acc = acc + all_v[v, pl.ds(k * L, L)]
            accn = accn + all_v[v, pl.ds(C + k * L, L)]
        hist.append(acc)
        cnt.append(accn)

    for q in range(RPS // L):
        idx = cc_acc[q]
        off = idx & (L - 1)
        chunk = idx >> 4
        g = hist[0][off]
        for k in range(1, C // L):
            g = jnp.where(chunk == k, hist[k][off], g)
        w_v[pl.ds(q * L, L)] = g

    pltpu.sync_copy(w_v, stats_hbm.at[0, pl.ds(base, RPS)])

    @pl.when(sid == 0)
    def _cnt_row():
        for k in range(C // L):
            cnt_v[pl.ds(k * L, L)] = cnt[k]
        pltpu.sync_copy(cnt_v, stats_hbm.at[1, pl.ds(0, C)])

    cout0.wait()
    cout1.wait()


def _sc_mine(labels, logits):
    mesh = plsc.VectorSubcoreMesh(core_axis_name="c", subcore_axis_name="s",
                                  num_cores=1)
    return pl.kernel(
        _sc_mine_body,
        out_type=[jax.ShapeDtypeStruct((B, C), jnp.float32),
                  jax.ShapeDtypeStruct((B, C), jnp.float32),
                  jax.ShapeDtypeStruct((8, B), jnp.float32)],
        mesh=mesh,
        scratch_types=[
            pltpu.VMEM((RPS, C), jnp.float32),
            pltpu.VMEM((RPS,), jnp.int32),
            pltpu.VMEM((RPS, C), jnp.float32),
            pltpu.VMEM((RPS, C), jnp.float32),
            pltpu.VMEM((RPS,), jnp.float32),
            pltpu.VMEM((C,), jnp.float32),
            pltpu.VMEM((2 * C,), jnp.float32),
            pltpu.VMEM((NSUB, 2 * C), jnp.float32),
            pltpu.VMEM_SHARED((NSUB, 2 * C), jnp.float32),
            pltpu.SemaphoreType.DMA,
        ],
    )(labels, logits)


def _nce_body(src_ref, tgt_ref, laboh_ref, ccoh_ref, stats_ref, out_ref,
              lnw_ref, acc_ref):
    i = pl.program_id(0)

    @pl.when(i == 0)
    def _prep():
        w = stats_ref[pl.ds(0, 1), :]
        lnw_ref[:, :] = jnp.where(w > 0.0, jnp.log(w), NEG_INF)
        acc_ref[0] = 0.0
        acc_ref[1] = 0.0
        acc_ref[2] = jnp.sum(w)

    src = src_ref[pl.ds(i * ROWS, ROWS), :]
    lab_oh = laboh_ref[pl.ds(i * ROWS, ROWS), :]
    s = lax.dot_general(src, tgt_ref[:], _NT,
                        preferred_element_type=jnp.float32)

    t = s + lnw_ref[:, :]
    smax = jnp.max(t, axis=1, keepdims=True)
    e = jnp.exp(t - smax)
    sumexp = jnp.sum(e, axis=1, keepdims=True)
    lse = smax + jnp.log(sumexp)

    G = lax.dot_general(s, ccoh_ref[:, :], _NN,
                        preferred_element_type=jnp.float32)
    cnt = stats_ref[pl.ds(1, 1), pl.ds(0, C)]
    rowcnt = lax.dot_general(lab_oh, cnt, _NT,
                             preferred_element_type=jnp.float32)

    acc_ref[0] += jnp.sum(lab_oh * G)
    acc_ref[1] += jnp.sum(jnp.where(rowcnt > 0.0, rowcnt * lse, 0.0))

    @pl.when(i == STEPS - 1)
    def _finish():
        nce = (acc_ref[0] - acc_ref[1]) / (-1.0 * B * acc_ref[2])
        out_ref[:, :] = jnp.reshape(nce, (1, 1))


@jax.jit
def kernel(src_feas, src_labels, tgt_feas, tgt_logits):
    labels = src_labels.astype(jnp.int32)
    lab_oh, cc_oh, stats = _sc_mine(labels, tgt_logits)

    out = pl.pallas_call(
        _nce_body,
        grid=(STEPS,),
        in_specs=[
            pl.BlockSpec((B, D), lambda i: (0, 0)),
            pl.BlockSpec((B, D), lambda i: (0, 0)),
            pl.BlockSpec((B, C), lambda i: (0, 0)),
            pl.BlockSpec((B, C), lambda i: (0, 0)),
            pl.BlockSpec((8, B), lambda i: (0, 0)),
        ],
        out_specs=pl.BlockSpec((1, 1), lambda i: (0, 0)),
        out_shape=jax.ShapeDtypeStruct((1, 1), jnp.float32),
        scratch_shapes=[
            pltpu.VMEM((1, B), jnp.float32),
            pltpu.SMEM((4,), jnp.float32),
        ],
    )(src_feas, tgt_feas, lab_oh, cc_oh, stats)
    return out[0, 0]
